# baseline (speedup 1.0000x reference)
import functools

import jax
import jax.numpy as jnp
from jax import lax
from jax.experimental import pallas as pl
from jax.experimental.pallas import tpu as pltpu
from jax.experimental.pallas import tpu_sc as plsc

_G = 64
_SC_PARAMS = pltpu.CompilerParams(needs_layout_passes=False)
_C = 2000
_CA = 8000
_CPT = 4
_NTILES = 32
_NC = 2


def _wid():
    return lax.axis_index("s") * _NC + lax.axis_index("c")


def _sc_deg_body(sd_hbm, ew_hbm, degp_hbm, acc, dv0, wv0, dv1, wv1,
                 sem0, sem1, *, N, E):
    wid = _wid()
    epp = E // _NTILES
    zero = jnp.zeros((16,), jnp.float32)
    bufs = [[dv0, wv0], [dv1, wv1]]

    def process(slot):
        sdv, ewv = bufs[slot]

        @plsc.parallel_loop(0, _C // 16, unroll=4)
        def ib(i):
            v16 = sdv[pl.ds(i * 16, 16)]
            d16 = lax.shift_right_logical(v16, 16)
            w16 = ewv[pl.ds(i * 16, 16)]
            plsc.addupdate_scatter(acc, [d16], w16)

    off0 = wid * epp
    pltpu.async_copy(sd_hbm.at[pl.ds(off0, _C)], dv0, sem0)
    pltpu.async_copy(ew_hbm.at[pl.ds(off0, _C)], wv0, sem0)

    @plsc.parallel_loop(0, N // 16, unroll=4)
    def zbody(i):
        acc[pl.ds(i * 16, 16)] = zero

    nch = epp // _C

    def fire(slot, cidx):
        off = off0 + cidx * _C
        d, w, s = (dv0, wv0, sem0) if slot == 0 else (dv1, wv1, sem1)
        pltpu.async_copy(sd_hbm.at[pl.ds(off, _C)], d, s)
        pltpu.async_copy(ew_hbm.at[pl.ds(off, _C)], w, s)

    def wait(slot):
        d, w, s = (dv0, wv0, sem0) if slot == 0 else (dv1, wv1, sem1)
        pltpu.make_async_copy(sd_hbm.at[pl.ds(0, _C)], d, s).wait()
        pltpu.make_async_copy(ew_hbm.at[pl.ds(0, _C)], w, s).wait()

    def pairbody(i, c):
        c0 = 2 * i
        fire(1, c0 + 1)
        wait(0)
        process(0)

        @pl.when(c0 + 2 < nch)
        def _():
            fire(0, c0 + 2)

        wait(1)
        process(1)
        return c

    lax.fori_loop(0, nch // 2, pairbody, 0)
    if nch % 2 == 1:
        wait(0)
        process(0)
    pltpu.sync_copy(acc, degp_hbm.at[wid])


def _sc_deg(sd, ew, N, E):
    mesh = plsc.VectorSubcoreMesh(core_axis_name="c", subcore_axis_name="s")
    body = functools.partial(_sc_deg_body, N=N, E=E)
    f = pl.kernel(
        body,
        out_type=jax.ShapeDtypeStruct((_NTILES, N), jnp.float32),
        mesh=mesh,
        compiler_params=_SC_PARAMS,
        scratch_types=[
            pltpu.VMEM((N,), jnp.float32),
            pltpu.VMEM((_C,), jnp.int32),
            pltpu.VMEM((_C,), jnp.float32),
            pltpu.VMEM((_C,), jnp.int32),
            pltpu.VMEM((_C,), jnp.float32),
            pltpu.SemaphoreType.DMA,
            pltpu.SemaphoreType.DMA,
        ],
    )
    return f(sd, ew)


def _sc_agg_body(u_hbm, sd_hbm, ew_hbm, y_hbm, scratches, *, N, E, d):
    ucols = scratches[:_CPT]
    accs = scratches[_CPT:2 * _CPT]
    ebufs = [scratches[2 * _CPT:2 * _CPT + 2],
             scratches[2 * _CPT + 2:2 * _CPT + 4]]
    usem, sem0, sem1 = scratches[2 * _CPT + 4:]
    sems = [sem0, sem1]
    ncg = d // _CPT
    P = _NTILES // ncg
    epp = E // P
    nch = epp // _CA
    wid = _wid()
    p = wid // ncg
    cg = wid % ncg
    ebase = p * epp
    hbms = [sd_hbm, ew_hbm]

    def fire(slot, cidx):
        off = ebase + cidx * _CA
        for h, b in zip(hbms, ebufs[slot]):
            pltpu.async_copy(h.at[pl.ds(off, _CA)], b, sems[slot])

    def wait(slot):
        for h, b in zip(hbms, ebufs[slot]):
            pltpu.make_async_copy(h.at[pl.ds(0, _CA)], b, sems[slot]).wait()

    def process(slot):
        sdv, ewv = ebufs[slot]

        @plsc.parallel_loop(0, _CA // 16, unroll=10)
        def ib(i):
            v16 = sdv[pl.ds(i * 16, 16)]
            s16 = lax.bitwise_and(v16, 0xFFFF)
            d16 = lax.shift_right_logical(v16, 16)
            w16 = ewv[pl.ds(i * 16, 16)]
            for r in range(_CPT):
                g = plsc.load_gather(ucols[r], [s16])
                plsc.addupdate_scatter(accs[r], [d16], g * w16)

    fire(0, 0)
    for r in range(_CPT):
        pltpu.async_copy(u_hbm.at[cg * _CPT + r], ucols[r], usem)

    zero = jnp.zeros((16,), jnp.float32)

    @plsc.parallel_loop(0, N // 16, unroll=4)
    def zbody(i):
        for r in range(_CPT):
            accs[r][pl.ds(i * 16, 16)] = zero

    for r in range(_CPT):
        pltpu.make_async_copy(u_hbm.at[cg * _CPT + r], ucols[r], usem).wait()

    def pairbody(i, c):
        c0 = 2 * i
        fire(1, c0 + 1)
        wait(0)
        process(0)

        @pl.when(c0 + 2 < nch)
        def _():
            fire(0, c0 + 2)

        wait(1)
        process(1)
        return c

    lax.fori_loop(0, nch // 2, pairbody, 0)
    if nch % 2 == 1:
        wait(0)
        process(0)

    for r in range(_CPT):
        pltpu.async_copy(accs[r], y_hbm.at[p, cg * _CPT + r], usem)
    for r in range(_CPT):
        pltpu.make_async_copy(accs[r], y_hbm.at[p, cg * _CPT + r], usem).wait()


def _sc_agg(u, sd, ew, N, E):
    d = u.shape[0]
    ncg = d // _CPT
    P = _NTILES // ncg
    mesh = plsc.VectorSubcoreMesh(core_axis_name="c", subcore_axis_name="s")

    def body(u_hbm, sd_hbm, ew_hbm, y_hbm, *scratches):
        _sc_agg_body(u_hbm, sd_hbm, ew_hbm, y_hbm, scratches,
                     N=N, E=E, d=d)

    f = pl.kernel(
        body,
        out_type=jax.ShapeDtypeStruct((P, d, N), jnp.float32),
        mesh=mesh,
        compiler_params=_SC_PARAMS,
        scratch_types=(
            [pltpu.VMEM((N,), jnp.float32) for _ in range(2 * _CPT)]
            + [pltpu.VMEM((_CA,), jnp.int32),
               pltpu.VMEM((_CA,), jnp.float32)] * 2
            + [pltpu.SemaphoreType.DMA] * 3
        ),
    )
    return f(u, sd, ew)


def _tc_h1_body(x_ref, w_ref, h_ref):
    h_ref[...] = lax.dot_general(w_ref[...], x_ref[...], (((0,), (1,)), ((), ())),
                                 preferred_element_type=jnp.float32)


def _tc_dis_body(degp_ref, h_ref, dis_ref, u_ref):
    deg = 1.0 + jnp.sum(degp_ref[...], axis=0, keepdims=True)
    dis = lax.rsqrt(deg)
    dis_ref[...] = dis
    u_ref[...] = dis * h_ref[...]


def _tc_combine_body(y_ref, u_ref, dis_ref, b_ref, out_ref):
    ysum = jnp.sum(y_ref[...], axis=0)
    a = dis_ref[...] * (ysum + u_ref[...])
    out_ref[...] = dis_ref[...] * jnp.maximum(a + b_ref[...], 0.0)


def _tc_layer_body(y_ref, u_ref, dis_ref, w_ref, b_ref, out_ref):
    ysum = jnp.sum(y_ref[...], axis=0)
    a = dis_ref[...] * (ysum + u_ref[...])
    h = lax.dot_general(w_ref[...], a, (((0,), (0,)), ((), ())),
                        preferred_element_type=jnp.float32)
    out_ref[...] = dis_ref[...] * jnp.maximum(h + b_ref[...], 0.0)


def _tc_final_body(y_ref, u_ref, dis_ref, w_ref, b_ref, batch_ref, out_ref):
    N = u_ref.shape[1]
    ysum = jnp.sum(y_ref[...], axis=0)
    a = dis_ref[...] * (ysum + u_ref[...])
    h = lax.dot_general(w_ref[...], a, (((0,), (0,)), ((), ())),
                        preferred_element_type=jnp.float32) + b_ref[...]
    gids = lax.broadcasted_iota(jnp.int32, (_G, N), 0)
    oh = (gids == batch_ref[...]).astype(jnp.float32)
    pooled = lax.dot_general(oh, h, (((1,), (1,)), ((), ())),
                             preferred_element_type=jnp.float32)
    out_ref[...] = jnp.maximum(pooled, 0.0)


def _tc_call(body, out_shape, *args):
    return pl.pallas_call(body, out_shape=out_shape)(*args)


def kernel(x, edge_index, edge_weight, batch, W1, b1, W2, b2, W3, b3,
           W4, b4, W5, b5):
    N = x.shape[0]
    E = edge_index.shape[1]
    f32 = jnp.float32
    src = edge_index[0]
    dst = edge_index[1]
    sd = jnp.bitwise_or(src, jnp.left_shift(dst, 16))
    b1c = jnp.reshape(b1, (-1, 1))
    b2c = jnp.reshape(b2, (-1, 1))
    b3c = jnp.reshape(b3, (-1, 1))
    b4c = jnp.reshape(b4, (-1, 1))
    b5c = jnp.reshape(b5, (-1, 1))
    batch2 = jnp.reshape(batch, (1, N))

    degp = _sc_deg(sd, edge_weight, N, E)
    h1 = _tc_call(_tc_h1_body, jax.ShapeDtypeStruct((16, N), f32), x, W1)
    dis, u1 = _tc_call(_tc_dis_body,
                       (jax.ShapeDtypeStruct((1, N), f32),
                        jax.ShapeDtypeStruct((16, N), f32)), degp, h1)

    y1 = _sc_agg(u1, sd, edge_weight, N, E)
    u2 = _tc_call(_tc_combine_body, jax.ShapeDtypeStruct((16, N), f32),
                  y1, u1, dis, b1c)

    y2 = _sc_agg(u2, sd, edge_weight, N, E)
    u3 = _tc_call(_tc_layer_body, jax.ShapeDtypeStruct((32, N), f32),
                  y2, u2, dis, W2, b2c)

    y3 = _sc_agg(u3, sd, edge_weight, N, E)
    u4 = _tc_call(_tc_layer_body, jax.ShapeDtypeStruct((64, N), f32),
                  y3, u3, dis, W3, b3c)

    y4 = _sc_agg(u4, sd, edge_weight, N, E)
    u5 = _tc_call(_tc_layer_body, jax.ShapeDtypeStruct((64, N), f32),
                  y4, u4, dis, W4, b4c)

    y5 = _sc_agg(u5, sd, edge_weight, N, E)
    out = _tc_call(_tc_final_body, jax.ShapeDtypeStruct((_G, 128), f32),
                   y5, u5, dis, W5, b5c, batch2)
    return out


# speedup vs baseline: 1.0313x; 1.0313x over previous
import functools

import jax
import jax.numpy as jnp
from jax import lax
from jax.experimental import pallas as pl
from jax.experimental.pallas import tpu as pltpu
from jax.experimental.pallas import tpu_sc as plsc

_G = 64
_SC_PARAMS = pltpu.CompilerParams(needs_layout_passes=False)
_C = 2000
_CA = 8000
_CPT = 4
_NTILES = 32
_NC = 2


def _wid():
    return lax.axis_index("s") * _NC + lax.axis_index("c")


def _sc_deg_body(sd_hbm, ew_hbm, degp_hbm, acc, dv0, wv0, dv1, wv1,
                 sem0, sem1, *, N, E):
    wid = _wid()
    epp = E // _NTILES
    zero = jnp.zeros((16,), jnp.float32)
    bufs = [[dv0, wv0], [dv1, wv1]]

    def process(slot):
        sdv, ewv = bufs[slot]

        @plsc.parallel_loop(0, _C // 16, unroll=4)
        def ib(i):
            v16 = sdv[pl.ds(i * 16, 16)]
            d16 = lax.shift_right_logical(v16, 16)
            w16 = ewv[pl.ds(i * 16, 16)]
            plsc.addupdate_scatter(acc, [d16], w16)

    off0 = wid * epp
    pltpu.async_copy(sd_hbm.at[pl.ds(off0, _C)], dv0, sem0)
    pltpu.async_copy(ew_hbm.at[pl.ds(off0, _C)], wv0, sem0)

    @plsc.parallel_loop(0, N // 16, unroll=4)
    def zbody(i):
        acc[pl.ds(i * 16, 16)] = zero

    nch = epp // _C

    def fire(slot, cidx):
        off = off0 + cidx * _C
        d, w, s = (dv0, wv0, sem0) if slot == 0 else (dv1, wv1, sem1)
        pltpu.async_copy(sd_hbm.at[pl.ds(off, _C)], d, s)
        pltpu.async_copy(ew_hbm.at[pl.ds(off, _C)], w, s)

    def wait(slot):
        d, w, s = (dv0, wv0, sem0) if slot == 0 else (dv1, wv1, sem1)
        pltpu.make_async_copy(sd_hbm.at[pl.ds(0, _C)], d, s).wait()
        pltpu.make_async_copy(ew_hbm.at[pl.ds(0, _C)], w, s).wait()

    def pairbody(i, c):
        c0 = 2 * i
        fire(1, c0 + 1)
        wait(0)
        process(0)

        @pl.when(c0 + 2 < nch)
        def _():
            fire(0, c0 + 2)

        wait(1)
        process(1)
        return c

    lax.fori_loop(0, nch // 2, pairbody, 0)
    if nch % 2 == 1:
        wait(0)
        process(0)
    pltpu.sync_copy(acc, degp_hbm.at[wid])


def _sc_deg(sd, ew, N, E):
    mesh = plsc.VectorSubcoreMesh(core_axis_name="c", subcore_axis_name="s")
    body = functools.partial(_sc_deg_body, N=N, E=E)
    f = pl.kernel(
        body,
        out_type=jax.ShapeDtypeStruct((_NTILES, N), jnp.float32),
        mesh=mesh,
        compiler_params=_SC_PARAMS,
        scratch_types=[
            pltpu.VMEM((N,), jnp.float32),
            pltpu.VMEM((_C,), jnp.int32),
            pltpu.VMEM((_C,), jnp.float32),
            pltpu.VMEM((_C,), jnp.int32),
            pltpu.VMEM((_C,), jnp.float32),
            pltpu.SemaphoreType.DMA,
            pltpu.SemaphoreType.DMA,
        ],
    )
    return f(sd, ew)


def _sc_agg_body(u_hbm, sd_hbm, ew_hbm, y_hbm, scratches, *, N, E, d):
    ucols = scratches[:_CPT]
    accs = scratches[_CPT:2 * _CPT]
    ebufs = [scratches[2 * _CPT:2 * _CPT + 2],
             scratches[2 * _CPT + 2:2 * _CPT + 4]]
    usem, sem0, sem1 = scratches[2 * _CPT + 4:]
    sems = [sem0, sem1]
    ncg = d // _CPT
    P = _NTILES // ncg
    epp = E // P
    nch = epp // _CA
    wid = _wid()
    p = wid // ncg
    cg = wid % ncg
    ebase = p * epp
    hbms = [sd_hbm, ew_hbm]

    def fire(slot, cidx):
        off = ebase + cidx * _CA
        for h, b in zip(hbms, ebufs[slot]):
            pltpu.async_copy(h.at[pl.ds(off, _CA)], b, sems[slot])

    def wait(slot):
        for h, b in zip(hbms, ebufs[slot]):
            pltpu.make_async_copy(h.at[pl.ds(0, _CA)], b, sems[slot]).wait()

    def process(slot):
        sdv, ewv = ebufs[slot]

        @plsc.parallel_loop(0, _CA // 16, unroll=5)
        def ib(i):
            v16 = sdv[pl.ds(i * 16, 16)]
            s16 = lax.bitwise_and(v16, 0xFFFF)
            d16 = lax.shift_right_logical(v16, 16)
            w16 = ewv[pl.ds(i * 16, 16)]
            for r in range(_CPT):
                g = plsc.load_gather(ucols[r], [s16])
                plsc.addupdate_scatter(accs[r], [d16], g * w16)

    fire(0, 0)
    for r in range(_CPT):
        pltpu.async_copy(u_hbm.at[cg * _CPT + r], ucols[r], usem)

    zero = jnp.zeros((16,), jnp.float32)

    @plsc.parallel_loop(0, N // 16, unroll=4)
    def zbody(i):
        for r in range(_CPT):
            accs[r][pl.ds(i * 16, 16)] = zero

    for r in range(_CPT):
        pltpu.make_async_copy(u_hbm.at[cg * _CPT + r], ucols[r], usem).wait()

    def pairbody(i, c):
        c0 = 2 * i
        fire(1, c0 + 1)
        wait(0)
        process(0)

        @pl.when(c0 + 2 < nch)
        def _():
            fire(0, c0 + 2)

        wait(1)
        process(1)
        return c

    lax.fori_loop(0, nch // 2, pairbody, 0)
    if nch % 2 == 1:
        wait(0)
        process(0)

    for r in range(_CPT):
        pltpu.async_copy(accs[r], y_hbm.at[p, cg * _CPT + r], usem)
    for r in range(_CPT):
        pltpu.make_async_copy(accs[r], y_hbm.at[p, cg * _CPT + r], usem).wait()


def _sc_agg(u, sd, ew, N, E):
    d = u.shape[0]
    ncg = d // _CPT
    P = _NTILES // ncg
    mesh = plsc.VectorSubcoreMesh(core_axis_name="c", subcore_axis_name="s")

    def body(u_hbm, sd_hbm, ew_hbm, y_hbm, *scratches):
        _sc_agg_body(u_hbm, sd_hbm, ew_hbm, y_hbm, scratches,
                     N=N, E=E, d=d)

    f = pl.kernel(
        body,
        out_type=jax.ShapeDtypeStruct((P, d, N), jnp.float32),
        mesh=mesh,
        compiler_params=_SC_PARAMS,
        scratch_types=(
            [pltpu.VMEM((N,), jnp.float32) for _ in range(2 * _CPT)]
            + [pltpu.VMEM((_CA,), jnp.int32),
               pltpu.VMEM((_CA,), jnp.float32)] * 2
            + [pltpu.SemaphoreType.DMA] * 3
        ),
    )
    return f(u, sd, ew)


def _tc_h1_body(x_ref, w_ref, h_ref):
    h_ref[...] = lax.dot_general(w_ref[...], x_ref[...], (((0,), (1,)), ((), ())),
                                 preferred_element_type=jnp.float32)


def _tc_dis_body(degp_ref, h_ref, dis_ref, u_ref):
    deg = 1.0 + jnp.sum(degp_ref[...], axis=0, keepdims=True)
    dis = lax.rsqrt(deg)
    dis_ref[...] = dis
    u_ref[...] = dis * h_ref[...]


def _tc_combine_body(y_ref, u_ref, dis_ref, b_ref, out_ref):
    ysum = jnp.sum(y_ref[...], axis=0)
    a = dis_ref[...] * (ysum + u_ref[...])
    out_ref[...] = dis_ref[...] * jnp.maximum(a + b_ref[...], 0.0)


def _tc_layer_body(y_ref, u_ref, dis_ref, w_ref, b_ref, out_ref):
    ysum = jnp.sum(y_ref[...], axis=0)
    a = dis_ref[...] * (ysum + u_ref[...])
    h = lax.dot_general(w_ref[...], a, (((0,), (0,)), ((), ())),
                        preferred_element_type=jnp.float32)
    out_ref[...] = dis_ref[...] * jnp.maximum(h + b_ref[...], 0.0)


def _tc_final_body(y_ref, u_ref, dis_ref, w_ref, b_ref, batch_ref, out_ref):
    N = u_ref.shape[1]
    ysum = jnp.sum(y_ref[...], axis=0)
    a = dis_ref[...] * (ysum + u_ref[...])
    h = lax.dot_general(w_ref[...], a, (((0,), (0,)), ((), ())),
                        preferred_element_type=jnp.float32) + b_ref[...]
    gids = lax.broadcasted_iota(jnp.int32, (_G, N), 0)
    oh = (gids == batch_ref[...]).astype(jnp.float32)
    pooled = lax.dot_general(oh, h, (((1,), (1,)), ((), ())),
                             preferred_element_type=jnp.float32)
    out_ref[...] = jnp.maximum(pooled, 0.0)


def _tc_call(body, out_shape, *args):
    return pl.pallas_call(body, out_shape=out_shape)(*args)


def kernel(x, edge_index, edge_weight, batch, W1, b1, W2, b2, W3, b3,
           W4, b4, W5, b5):
    N = x.shape[0]
    E = edge_index.shape[1]
    f32 = jnp.float32
    src = edge_index[0]
    dst = edge_index[1]
    sd = jnp.bitwise_or(src, jnp.left_shift(dst, 16))
    b1c = jnp.reshape(b1, (-1, 1))
    b2c = jnp.reshape(b2, (-1, 1))
    b3c = jnp.reshape(b3, (-1, 1))
    b4c = jnp.reshape(b4, (-1, 1))
    b5c = jnp.reshape(b5, (-1, 1))
    batch2 = jnp.reshape(batch, (1, N))

    degp = _sc_deg(sd, edge_weight, N, E)
    h1 = _tc_call(_tc_h1_body, jax.ShapeDtypeStruct((16, N), f32), x, W1)
    dis, u1 = _tc_call(_tc_dis_body,
                       (jax.ShapeDtypeStruct((1, N), f32),
                        jax.ShapeDtypeStruct((16, N), f32)), degp, h1)

    y1 = _sc_agg(u1, sd, edge_weight, N, E)
    u2 = _tc_call(_tc_combine_body, jax.ShapeDtypeStruct((16, N), f32),
                  y1, u1, dis, b1c)

    y2 = _sc_agg(u2, sd, edge_weight, N, E)
    u3 = _tc_call(_tc_layer_body, jax.ShapeDtypeStruct((32, N), f32),
                  y2, u2, dis, W2, b2c)

    y3 = _sc_agg(u3, sd, edge_weight, N, E)
    u4 = _tc_call(_tc_layer_body, jax.ShapeDtypeStruct((64, N), f32),
                  y3, u3, dis, W3, b3c)

    y4 = _sc_agg(u4, sd, edge_weight, N, E)
    u5 = _tc_call(_tc_layer_body, jax.ShapeDtypeStruct((64, N), f32),
                  y4, u4, dis, W4, b4c)

    y5 = _sc_agg(u5, sd, edge_weight, N, E)
    out = _tc_call(_tc_final_body, jax.ShapeDtypeStruct((_G, 128), f32),
                   y5, u5, dis, W5, b5c, batch2)
    return out
